# baseline (speedup 1.0000x reference)
import functools

import jax
import jax.numpy as jnp
from jax import lax
from jax.experimental import pallas as pl
from jax.experimental.pallas import tpu as pltpu
from jax.experimental.pallas import tpu_sc as plsc

B = 512
IN = 512
L1 = 512
ED = 64
D = 128
K = 1024
CC = 0.25
N = B * ED

NBLK = 8
BB = B // NBLK
FB = BB * ED

SC_CORES = 2
SC_SUBCORES = 16
NW = SC_CORES * SC_SUBCORES
ROWS_PW = N // NW
CH = 128
NCH = ROWS_PW // CH


def _enc_dist_body(x_ref, eW1_ref, eb1_ref, eW2_ref, eb2_ref,
                   ersW1_ref, ersb1_ref, ersW2_ref, ersb2_ref,
                   ebn_g_ref, ebn_b_ref, cb_ref, teW1_ref, teb1_ref,
                   teW2_ref, teb2_ref,
                   z_ref, idx_ref, counts_ref, sse_ref):
    relu = lambda v: jnp.maximum(v, 0.0)
    xb = x_ref[...]
    z1 = relu(jnp.dot(xb, eW1_ref[...], preferred_element_type=jnp.float32)
              + eb1_ref[...][None, :])
    z2 = relu(jnp.dot(z1, eW2_ref[...], preferred_element_type=jnp.float32)
              + eb2_ref[...][None, :])
    h = relu(jnp.dot(z2, ersW1_ref[...], preferred_element_type=jnp.float32)
             + ersb1_ref[...][None, :])
    h = relu(jnp.dot(h, ersW2_ref[...], preferred_element_type=jnp.float32)
             + ersb2_ref[...][None, :])
    h = (ebn_g_ref[...][None, :] * h / jnp.sqrt(1.0 + 1e-5)
         + ebn_b_ref[...][None, :])
    zb = z2 + h
    z_ref[...] = zb
    e1 = relu(zb[:, :, None] * teW1_ref[0][None, None, :]
              + teb1_ref[...][None, None, :])
    eT = (lax.dot_general(teW2_ref[...], e1.reshape(FB, D // 2),
                          (((0,), (1,)), ((), ())),
                          preferred_element_type=jnp.float32)
          + teb2_ref[...][:, None])
    cb = cb_ref[...]
    fnormT = jnp.sum(eT * eT, axis=0, keepdims=True)
    cnormc = jnp.sum(cb * cb, axis=1, keepdims=True)
    mmT = lax.dot_general(cb, eT, (((1,), (0,)), ((), ())),
                          preferred_element_type=jnp.float32)
    distT = fnormT + cnormc - 2.0 * mmT
    rowminT = jnp.min(distT, axis=0, keepdims=True)
    eqT = distT == rowminT
    kiota = lax.broadcasted_iota(jnp.int32, (K, FB), 0)
    idxrow = jnp.min(jnp.where(eqT, kiota, K), axis=0, keepdims=True)
    idx_ref[...] = idxrow[None]
    cpartc = jnp.sum(eqT.astype(jnp.float32), axis=1, keepdims=True)
    ssepart = jnp.sum(rowminT)
    @pl.when(pl.program_id(0) == 0)
    def _init():
        counts_ref[...] = cpartc
        sse_ref[0, 0] = ssepart
    @pl.when(pl.program_id(0) != 0)
    def _acc():
        counts_ref[...] += cpartc
        sse_ref[0, 0] += ssepart


def _stage1(x, eW1, eb1, eW2, eb2, ersW1, ersb1, ersW2, ersb2, ebn_g, ebn_b,
            codebook, teW1, teb1, teW2, teb2):
    full = lambda shape: pl.BlockSpec(shape, lambda i: (0,) * len(shape))
    return pl.pallas_call(
        _enc_dist_body,
        grid=(NBLK,),
        in_specs=[
            pl.BlockSpec((BB, IN), lambda i: (i, 0)),
            full((IN, L1)), full((L1,)), full((L1, ED)), full((ED,)),
            full((ED, 16)), full((16,)), full((16, ED)), full((ED,)),
            full((ED,)), full((ED,)), full((K, D)),
            full((1, D // 2)), full((D // 2,)), full((D // 2, D)),
            full((D,)),
        ],
        out_specs=[
            pl.BlockSpec((BB, ED), lambda i: (i, 0)),
            pl.BlockSpec((1, 1, FB), lambda i: (i, 0, 0)),
            pl.BlockSpec((K, 1), lambda i: (0, 0)),
            pl.BlockSpec(memory_space=pltpu.SMEM),
        ],
        out_shape=[
            jax.ShapeDtypeStruct((B, ED), jnp.float32),
            jax.ShapeDtypeStruct((NBLK, 1, FB), jnp.int32),
            jax.ShapeDtypeStruct((K, 1), jnp.float32),
            jax.ShapeDtypeStruct((1, 1), jnp.float32),
        ],
        compiler_params=pltpu.CompilerParams(
            dimension_semantics=("arbitrary",)),
    )(x, eW1, eb1, eW2, eb2, ersW1, ersb1, ersW2, ersb2,
      ebn_g, ebn_b, codebook, teW1, teb1, teW2, teb2)


def _sc_gather(codebook, idx):
    mesh = plsc.VectorSubcoreMesh(core_axis_name="c", subcore_axis_name="s",
                                  num_cores=SC_CORES, num_subcores=SC_SUBCORES)
    stage = K // SC_SUBCORES

    @functools.partial(
        pl.kernel,
        out_type=jax.ShapeDtypeStruct((N, D), jnp.float32),
        mesh=mesh,
        scratch_types=[
            pltpu.VMEM((ROWS_PW,), jnp.int32),
            pltpu.VMEM((2, CH, D), jnp.float32),
            pltpu.VMEM_SHARED((K, D), jnp.float32),
            pltpu.SemaphoreType.DMA,
            pltpu.SemaphoreType.DMA,
            pltpu.SemaphoreType.DMA,
            pltpu.SemaphoreType.DMA,
        ],
    )
    def gather(cb_hbm, idx_hbm, out_hbm, idx_v, rows_v, cb_sh,
               gsem0, gsem1, osem0, osem1):
        cid = lax.axis_index("c")
        sid = lax.axis_index("s")
        wid = sid * SC_CORES + cid
        base = wid * ROWS_PW
        pltpu.sync_copy(cb_hbm.at[pl.ds(sid * stage, stage)],
                        cb_sh.at[pl.ds(sid * stage, stage)])
        pltpu.sync_copy(idx_hbm.at[pl.ds(base, ROWS_PW)], idx_v)
        plsc.subcore_barrier()
        gsems = (gsem0, gsem1)
        osems = (osem0, osem1)
        ghandles = [None, None]
        ohandles = [None, None]
        ghandles[0] = pltpu.async_copy(
            cb_sh.at[idx_v.at[pl.ds(0, CH)]], rows_v.at[0], gsems[0])
        for c in range(NCH):
            b = c % 2
            nb = (c + 1) % 2
            ghandles[b].wait()
            if c + 1 < NCH:
                if ohandles[nb] is not None:
                    ohandles[nb].wait()
                ghandles[nb] = pltpu.async_copy(
                    cb_sh.at[idx_v.at[pl.ds((c + 1) * CH, CH)]],
                    rows_v.at[nb], gsems[nb])
            ohandles[b] = pltpu.async_copy(
                rows_v.at[b], out_hbm.at[pl.ds(base + c * CH, CH)], osems[b])
        ohandles[0].wait()
        ohandles[1].wait()

    return gather(codebook, idx)


def _tail_body(q_ref, counts_ref, sse_ref, dpW_ref, dpb_ref,
               drsW1_ref, drsb1_ref, drsW2_ref, drsb2_ref,
               dbn_g_ref, dbn_b_ref, dl1W_ref, dl1b_ref, dl2W_ref, dl2b_ref,
               clsW_ref, clsb_ref,
               q2_ref, xr_ref, cls_ref, vq_ref, perp_ref):
    relu = lambda v: jnp.maximum(v, 0.0)
    m = sse_ref[0, 0] / float(N * D)
    vq_ref[0, 0] = m + CC * m
    avg = counts_ref[...] / float(N)
    perp_ref[0, 0] = jnp.exp(-jnp.sum(avg * jnp.log(avg + 1e-10)))
    q2 = q_ref[...].reshape(B, ED * D)
    q2_ref[...] = q2
    cls_ref[...] = jax.nn.sigmoid(
        jnp.dot(q2, clsW_ref[...], preferred_element_type=jnp.float32)
        + clsb_ref[...][None, :])
    d1 = relu(jnp.dot(q2, dpW_ref[...], preferred_element_type=jnp.float32)
              + dpb_ref[...][None, :])
    h2 = relu(jnp.dot(d1, drsW1_ref[...], preferred_element_type=jnp.float32)
              + drsb1_ref[...][None, :])
    h2 = relu(jnp.dot(h2, drsW2_ref[...], preferred_element_type=jnp.float32)
              + drsb2_ref[...][None, :])
    h2 = (dbn_g_ref[...][None, :] * h2 / jnp.sqrt(1.0 + 1e-5)
          + dbn_b_ref[...][None, :])
    dd = d1 + h2
    r1 = relu(jnp.dot(dd, dl1W_ref[...], preferred_element_type=jnp.float32)
              + dl1b_ref[...][None, :])
    xr_ref[...] = relu(jnp.dot(r1, dl2W_ref[...],
                               preferred_element_type=jnp.float32)
                       + dl2b_ref[...][None, :])


def _stage3(quant, counts, sse, dpW, dpb, drsW1, drsb1, drsW2, drsb2,
            dbn_g, dbn_b, dl1W, dl1b, dl2W, dl2b, clsW, clsb):
    return pl.pallas_call(
        _tail_body,
        out_shape=[
            jax.ShapeDtypeStruct((B, ED * D), jnp.float32),
            jax.ShapeDtypeStruct((B, IN), jnp.float32),
            jax.ShapeDtypeStruct((B, 10), jnp.float32),
            jax.ShapeDtypeStruct((1, 1), jnp.float32),
            jax.ShapeDtypeStruct((1, 1), jnp.float32),
        ],
        in_specs=[pl.BlockSpec(memory_space=pltpu.VMEM)] * 2
        + [pl.BlockSpec(memory_space=pltpu.SMEM)]
        + [pl.BlockSpec(memory_space=pltpu.VMEM)] * 14,
        out_specs=[
            pl.BlockSpec(memory_space=pltpu.VMEM),
            pl.BlockSpec(memory_space=pltpu.VMEM),
            pl.BlockSpec(memory_space=pltpu.VMEM),
            pl.BlockSpec(memory_space=pltpu.SMEM),
            pl.BlockSpec(memory_space=pltpu.SMEM),
        ],
    )(quant, counts, sse, dpW, dpb, drsW1, drsb1, drsW2, drsb2,
      dbn_g, dbn_b, dl1W, dl1b, dl2W, dl2b, clsW, clsb)


def kernel(x, eW1, eb1, eW2, eb2, ersW1, ersb1, ersW2, ersb2, ebn_g, ebn_b,
           codebook, teW1, teb1, teW2, teb2, clsW, clsb,
           dpW, dpb, drsW1, drsb1, drsW2, drsb2, dbn_g, dbn_b,
           dl1W, dl1b, dl2W, dl2b):
    z, idx3, counts, sse = _stage1(
        x, eW1, eb1, eW2, eb2, ersW1, ersb1, ersW2, ersb2, ebn_g, ebn_b,
        codebook, teW1, teb1, teW2, teb2)
    idx = idx3.reshape(N)
    quant = _sc_gather(codebook, idx)
    q2, x_recon, cls, vq, perp = _stage3(
        quant, counts, sse, dpW, dpb, drsW1, drsb1, drsW2, drsb2,
        dbn_g, dbn_b, dl1W, dl1b, dl2W, dl2b, clsW, clsb)
    return (vq.reshape(()), x_recon, perp.reshape(()), q2, cls, z)


# speedup vs baseline: 1.0903x; 1.0903x over previous
import functools

import jax
import jax.numpy as jnp
from jax import lax
from jax.experimental import pallas as pl
from jax.experimental.pallas import tpu as pltpu
from jax.experimental.pallas import tpu_sc as plsc

B = 512
IN = 512
L1 = 512
ED = 64
D = 128
K = 1024
CC = 0.25
N = B * ED

NBLK = 8
BB = B // NBLK
FB = BB * ED

SC_CORES = 2
SC_SUBCORES = 16
NW = SC_CORES * SC_SUBCORES
ROWS_PW = N // NW
CH = 128
NCH = ROWS_PW // CH


def _enc_dist_body(x_ref, eW1_ref, eb1_ref, eW2t_ref, eb2_ref,
                   ersW1t_ref, ersb1_ref, ersW2_ref, ersb2_ref,
                   ebn_g_ref, ebn_b_ref, cb_ref, teW1_ref, teb1_ref,
                   teW2_ref, teb2_ref,
                   z_ref, idx_ref, counts_ref, sse_ref):
    relu = lambda v: jnp.maximum(v, 0.0)
    xb = x_ref[...]
    z1 = relu(jnp.dot(xb, eW1_ref[...], preferred_element_type=jnp.float32)
              + eb1_ref[...][None, :])
    z2 = relu(lax.dot_general(z1, eW2t_ref[...], (((1,), (1,)), ((), ())),
                              preferred_element_type=jnp.float32)
              + eb2_ref[...][None, :])
    h = relu(lax.dot_general(z2, ersW1t_ref[...], (((1,), (1,)), ((), ())),
                             preferred_element_type=jnp.float32)
             + ersb1_ref[...][None, :])
    h = relu(jnp.dot(h, ersW2_ref[...], preferred_element_type=jnp.float32)
             + ersb2_ref[...][None, :])
    h = (ebn_g_ref[...][None, :] * h / jnp.sqrt(1.0 + 1e-5)
         + ebn_b_ref[...][None, :])
    zb = z2 + h
    z_ref[...] = zb
    e1 = relu(zb[:, :, None] * teW1_ref[0][None, None, :]
              + teb1_ref[...][None, None, :])
    e = (jnp.dot(e1.reshape(FB, D // 2), teW2_ref[...],
                 preferred_element_type=jnp.float32)
         + teb2_ref[...][None, :])
    cb = cb_ref[...]
    fnorm = jnp.sum(e * e, axis=1, keepdims=True)
    cnorm = jnp.sum(cb * cb, axis=1)
    mm = lax.dot_general(e, cb, (((1,), (1,)), ((), ())),
                         preferred_element_type=jnp.float32)
    dist = fnorm + cnorm[None, :] - 2.0 * mm
    rowmin = jnp.min(dist, axis=1, keepdims=True)
    eq = dist == rowmin
    kiota = lax.broadcasted_iota(jnp.int32, (FB, K), 1)
    idxcol = jnp.min(jnp.where(eq, kiota, K), axis=1, keepdims=True)
    idx_ref[...] = jnp.transpose(idxcol)[None]
    cpartc = jnp.sum(eq.astype(jnp.float32), axis=0, keepdims=True)
    ssepart = jnp.sum(rowmin)
    @pl.when(pl.program_id(0) == 0)
    def _init():
        counts_ref[...] = cpartc
        sse_ref[0, 0] = ssepart
    @pl.when(pl.program_id(0) != 0)
    def _acc():
        counts_ref[...] += cpartc
        sse_ref[0, 0] += ssepart


def _stage1(x, eW1, eb1, eW2, eb2, ersW1, ersb1, ersW2, ersb2, ebn_g, ebn_b,
            codebook, teW1, teb1, teW2, teb2):
    full = lambda shape: pl.BlockSpec(shape, lambda i: (0,) * len(shape))
    return pl.pallas_call(
        _enc_dist_body,
        grid=(NBLK,),
        in_specs=[
            pl.BlockSpec((BB, IN), lambda i: (i, 0)),
            full((IN, L1)), full((L1,)), full((ED, L1)), full((ED,)),
            full((16, ED)), full((16,)), full((16, ED)), full((ED,)),
            full((ED,)), full((ED,)), full((K, D)),
            full((1, D // 2)), full((D // 2,)), full((D // 2, D)),
            full((D,)),
        ],
        out_specs=[
            pl.BlockSpec((BB, ED), lambda i: (i, 0)),
            pl.BlockSpec((1, 1, FB), lambda i: (i, 0, 0)),
            pl.BlockSpec((1, K), lambda i: (0, 0)),
            pl.BlockSpec(memory_space=pltpu.SMEM),
        ],
        out_shape=[
            jax.ShapeDtypeStruct((B, ED), jnp.float32),
            jax.ShapeDtypeStruct((NBLK, 1, FB), jnp.int32),
            jax.ShapeDtypeStruct((1, K), jnp.float32),
            jax.ShapeDtypeStruct((1, 1), jnp.float32),
        ],
        compiler_params=pltpu.CompilerParams(
            dimension_semantics=("arbitrary",)),
    )(x, eW1, eb1, jnp.transpose(eW2), eb2, jnp.transpose(ersW1),
      ersb1, ersW2, ersb2, ebn_g, ebn_b, codebook, teW1, teb1, teW2, teb2)


def _sc_gather(codebook, idx):
    mesh = plsc.VectorSubcoreMesh(core_axis_name="c", subcore_axis_name="s",
                                  num_cores=SC_CORES, num_subcores=SC_SUBCORES)
    stage = K // SC_SUBCORES

    @functools.partial(
        pl.kernel,
        out_type=jax.ShapeDtypeStruct((N, D), jnp.float32),
        mesh=mesh,
        scratch_types=[
            pltpu.VMEM((ROWS_PW,), jnp.int32),
            pltpu.VMEM((2, CH, D), jnp.float32),
            pltpu.VMEM_SHARED((K, D), jnp.float32),
            pltpu.SemaphoreType.DMA,
            pltpu.SemaphoreType.DMA,
            pltpu.SemaphoreType.DMA,
            pltpu.SemaphoreType.DMA,
        ],
    )
    def gather(cb_hbm, idx_hbm, out_hbm, idx_v, rows_v, cb_sh,
               gsem0, gsem1, osem0, osem1):
        cid = lax.axis_index("c")
        sid = lax.axis_index("s")
        wid = sid * SC_CORES + cid
        base = wid * ROWS_PW
        pltpu.sync_copy(cb_hbm.at[pl.ds(sid * stage, stage)],
                        cb_sh.at[pl.ds(sid * stage, stage)])
        pltpu.sync_copy(idx_hbm.at[pl.ds(base, ROWS_PW)], idx_v)
        plsc.subcore_barrier()
        gsems = (gsem0, gsem1)
        osems = (osem0, osem1)
        ghandles = [None, None]
        ohandles = [None, None]
        ghandles[0] = pltpu.async_copy(
            cb_sh.at[idx_v.at[pl.ds(0, CH)]], rows_v.at[0], gsems[0])
        for c in range(NCH):
            b = c % 2
            nb = (c + 1) % 2
            ghandles[b].wait()
            if c + 1 < NCH:
                if ohandles[nb] is not None:
                    ohandles[nb].wait()
                ghandles[nb] = pltpu.async_copy(
                    cb_sh.at[idx_v.at[pl.ds((c + 1) * CH, CH)]],
                    rows_v.at[nb], gsems[nb])
            ohandles[b] = pltpu.async_copy(
                rows_v.at[b], out_hbm.at[pl.ds(base + c * CH, CH)], osems[b])
        ohandles[0].wait()
        ohandles[1].wait()

    return gather(codebook, idx)


def _tail_body(q_ref, counts_ref, sse_ref, dpWt_ref, dpb_ref,
               drsW1t_ref, drsb1_ref, drsW2_ref, drsb2_ref,
               dbn_g_ref, dbn_b_ref, dl1W_ref, dl1b_ref, dl2W_ref, dl2b_ref,
               clsWt_ref, clsb_ref,
               q2_ref, xr_ref, clsT_ref, vq_ref, perp_ref):
    relu = lambda v: jnp.maximum(v, 0.0)
    m = sse_ref[0, 0] / float(N * D)
    vq_ref[0, 0] = m + CC * m
    avg = counts_ref[...] / float(N)
    perp_ref[0, 0] = jnp.exp(-jnp.sum(avg * jnp.log(avg + 1e-10)))
    q2 = q_ref[...].reshape(B, ED * D)
    q2_ref[...] = q2
    clsT_ref[...] = jax.nn.sigmoid(
        lax.dot_general(clsWt_ref[...], q2, (((1,), (1,)), ((), ())),
                        preferred_element_type=jnp.float32)
        + clsb_ref[...][:, None])
    d1 = relu(lax.dot_general(q2, dpWt_ref[...], (((1,), (1,)), ((), ())),
              preferred_element_type=jnp.float32)
              + dpb_ref[...][None, :])
    h2 = relu(lax.dot_general(d1, drsW1t_ref[...], (((1,), (1,)), ((), ())),
              preferred_element_type=jnp.float32)
              + drsb1_ref[...][None, :])
    h2 = relu(jnp.dot(h2, drsW2_ref[...], preferred_element_type=jnp.float32)
              + drsb2_ref[...][None, :])
    h2 = (dbn_g_ref[...][None, :] * h2 / jnp.sqrt(1.0 + 1e-5)
          + dbn_b_ref[...][None, :])
    dd = d1 + h2
    r1 = relu(jnp.dot(dd, dl1W_ref[...], preferred_element_type=jnp.float32)
              + dl1b_ref[...][None, :])
    xr_ref[...] = relu(jnp.dot(r1, dl2W_ref[...],
                               preferred_element_type=jnp.float32)
                       + dl2b_ref[...][None, :])


def _stage3(quant, counts, sse, dpW, dpb, drsW1, drsb1, drsW2, drsb2,
            dbn_g, dbn_b, dl1W, dl1b, dl2W, dl2b, clsW, clsb):
    return pl.pallas_call(
        _tail_body,
        out_shape=[
            jax.ShapeDtypeStruct((B, ED * D), jnp.float32),
            jax.ShapeDtypeStruct((B, IN), jnp.float32),
            jax.ShapeDtypeStruct((10, B), jnp.float32),
            jax.ShapeDtypeStruct((1, 1), jnp.float32),
            jax.ShapeDtypeStruct((1, 1), jnp.float32),
        ],
        in_specs=[pl.BlockSpec(memory_space=pltpu.VMEM)] * 2
        + [pl.BlockSpec(memory_space=pltpu.SMEM)]
        + [pl.BlockSpec(memory_space=pltpu.VMEM)] * 14,
        out_specs=[
            pl.BlockSpec(memory_space=pltpu.VMEM),
            pl.BlockSpec(memory_space=pltpu.VMEM),
            pl.BlockSpec(memory_space=pltpu.VMEM),
            pl.BlockSpec(memory_space=pltpu.SMEM),
            pl.BlockSpec(memory_space=pltpu.SMEM),
        ],
    )(quant, counts, sse, jnp.transpose(dpW), dpb, jnp.transpose(drsW1),
      drsb1, drsW2, drsb2, dbn_g, dbn_b, dl1W, dl1b, dl2W, dl2b,
      jnp.transpose(clsW), clsb)


def kernel(x, eW1, eb1, eW2, eb2, ersW1, ersb1, ersW2, ersb2, ebn_g, ebn_b,
           codebook, teW1, teb1, teW2, teb2, clsW, clsb,
           dpW, dpb, drsW1, drsb1, drsW2, drsb2, dbn_g, dbn_b,
           dl1W, dl1b, dl2W, dl2b):
    z, idx3, counts, sse = _stage1(
        x, eW1, eb1, eW2, eb2, ersW1, ersb1, ersW2, ersb2, ebn_g, ebn_b,
        codebook, teW1, teb1, teW2, teb2)
    idx = idx3.reshape(N)
    quant = _sc_gather(codebook, idx)
    q2, x_recon, clsT, vq, perp = _stage3(
        quant, counts, sse, dpW, dpb, drsW1, drsb1, drsW2, drsb2,
        dbn_g, dbn_b, dl1W, dl1b, dl2W, dl2b, clsW, clsb)
    return (vq.reshape(()), x_recon, perp.reshape(()), q2,
            jnp.transpose(clsT), z)
